# baseline (speedup 1.0000x reference)
import functools

import jax
import jax.numpy as jnp
from jax.experimental import pallas as pl
from jax.experimental.pallas import tpu as pltpu

B, N, DIM = 1, 2048, 2048
H, KVH, DH = 16, 4, 128
G = H // KVH
CBS = 32
SBS = 32
NSEL = 16
WIN = 64
W = N // CBS
CD = CBS * DH
TQ = 128
NT = N // TQ
HT = 512
NHT = CD // HT
SCALE = DH ** -0.5


def _proj_body(x_ref, g_ref, wq_ref, wk_ref, wv_ref, wc_ref, bc_ref,
               q_ref, k_ref, v_ref, st_ref):
    xt = x_ref[...]
    ms = jnp.mean(xt * xt, axis=-1, keepdims=True)
    xn = xt * jax.lax.rsqrt(ms + 1e-6) * g_ref[...]
    q_ref[...] = jnp.dot(xn, wq_ref[...], preferred_element_type=jnp.float32)
    k_ref[...] = jnp.dot(xn, wk_ref[...], preferred_element_type=jnp.float32)
    v_ref[...] = jnp.dot(xn, wv_ref[...], preferred_element_type=jnp.float32)
    st_ref[...] = jax.nn.sigmoid(
        jnp.dot(xn, wc_ref[...], preferred_element_type=jnp.float32)
        + bc_ref[...])


def _compress_body(kb_ref, kw1_ref, kb1_ref, kw2_ref, kb2_ref,
                   vb_ref, vw1_ref, vb1_ref, vw2_ref, vb2_ref,
                   ck_ref, cv_ref):
    j = pl.program_id(0)

    @pl.when(j == 0)
    def _():
        ck_ref[...] = jnp.broadcast_to(kb2_ref[...], ck_ref.shape)
        cv_ref[...] = jnp.broadcast_to(vb2_ref[...], cv_ref.shape)

    hk = jnp.maximum(
        jnp.dot(kb_ref[...], kw1_ref[...], preferred_element_type=jnp.float32)
        + kb1_ref[...], 0.0)
    ck_ref[...] += jnp.dot(hk, kw2_ref[...], preferred_element_type=jnp.float32)
    hv = jnp.maximum(
        jnp.dot(vb_ref[...], vw1_ref[...], preferred_element_type=jnp.float32)
        + vb1_ref[...], 0.0)
    cv_ref[...] += jnp.dot(hv, vw2_ref[...], preferred_element_type=jnp.float32)


def _cattn_body(q_ref, ck_ref, cv_ref, co_ref, imp_ref):
    i = pl.program_id(1)
    qt = q_ref[...].reshape(G * TQ, DH)
    ck = ck_ref[...].reshape(W + 1, DH)
    cv = cv_ref[...].reshape(W + 1, DH)
    sim = jax.lax.dot_general(qt, ck, (((1,), (1,)), ((), ())),
                              preferred_element_type=jnp.float32) * SCALE
    qp = i * TQ + jax.lax.rem(
        jax.lax.broadcasted_iota(jnp.int32, (G * TQ, W + 1), 0), TQ)
    col = jax.lax.broadcasted_iota(jnp.int32, (G * TQ, W + 1), 1)
    vis = qp >= col * CBS - 1
    s = jnp.where(vis, sim, -1e30)
    m = jnp.max(s, axis=-1, keepdims=True)
    e = jnp.exp(s - m)
    p = e / jnp.sum(e, axis=-1, keepdims=True)
    co_ref[...] = jax.lax.dot_general(
        p.astype(jnp.bfloat16), cv.astype(jnp.bfloat16),
        (((1,), (0,)), ((), ())),
        preferred_element_type=jnp.float32).reshape(1, G, TQ, DH)
    imp_ref[...] = jnp.mean(p[:, 1:].reshape(G, TQ, W),
                            axis=0).reshape(1, TQ, W)


def _sel_body(imp_ref, sel_ref):
    imp = imp_ref[...].reshape(N, W)
    cur = imp
    for _ in range(NSEL - 1):
        mx = jnp.max(cur, axis=-1, keepdims=True)
        cur = jnp.where(cur >= mx, -1.0, cur)
    thr = jnp.max(cur, axis=-1, keepdims=True)
    qp = jax.lax.broadcasted_iota(jnp.int32, (N, W), 0)
    blk = jax.lax.broadcasted_iota(jnp.int32, (N, W), 1)
    sel = (imp >= thr) | (blk == qp // SBS)
    sel_ref[...] = sel.astype(jnp.float32).reshape(1, N, W)


def _attn_body(q_ref, k_ref, v_ref, sel_ref, em_ref, fo_ref, so_ref,
               *, tile0, nw):
    i = tile0 + pl.program_id(1)
    wb = nw // CBS
    selb = sel_ref[...].reshape(TQ, W)[:, :wb]
    selt = jax.lax.dot_general(selb, em_ref[...], (((1,), (0,)), ((), ())),
                               preferred_element_type=jnp.float32) > 0.5
    qp = i * TQ + jax.lax.broadcasted_iota(jnp.int32, (TQ, nw), 0)
    jp = jax.lax.broadcasted_iota(jnp.int32, (TQ, nw), 1)
    causal = qp >= jp
    fmask = selt & causal
    cs = jnp.maximum(i - 1, 0) * TQ
    qps = i * TQ + jax.lax.broadcasted_iota(jnp.int32, (TQ, 2 * TQ), 0)
    jps = cs + jax.lax.broadcasted_iota(jnp.int32, (TQ, 2 * TQ), 1)
    smask = (qps >= jps) & (qps - jps <= WIN)
    ksw = k_ref[0, pl.ds(cs, 2 * TQ), :].astype(jnp.bfloat16)
    vsw = v_ref[0, pl.ds(cs, 2 * TQ), :].astype(jnp.bfloat16)
    kt = k_ref[...].reshape(nw, DH).astype(jnp.bfloat16)
    vt = v_ref[...].reshape(nw, DH).astype(jnp.bfloat16)
    for g in range(G):
        qg = q_ref[0, g].astype(jnp.bfloat16)
        s = jax.lax.dot_general(qg, kt, (((1,), (1,)), ((), ())),
                                preferred_element_type=jnp.float32) * SCALE
        fs = jnp.where(fmask, s, -1e30)
        fm = jnp.max(fs, axis=-1, keepdims=True)
        fe = jnp.exp(fs - fm)
        fl = jnp.sum(fe, axis=-1, keepdims=True)
        fo_ref[0, g] = jax.lax.dot_general(
            fe.astype(jnp.bfloat16), vt, (((1,), (0,)), ((), ())),
            preferred_element_type=jnp.float32) / fl
        ssw = jax.lax.dot_general(qg, ksw, (((1,), (1,)), ((), ())),
                                  preferred_element_type=jnp.float32) * SCALE
        ss = jnp.where(smask, ssw, -1e30)
        sm = jnp.max(ss, axis=-1, keepdims=True)
        se = jnp.exp(ss - sm)
        sl = jnp.sum(se, axis=-1, keepdims=True)
        so_ref[0, g] = jax.lax.dot_general(
            se.astype(jnp.bfloat16), vsw, (((1,), (0,)), ((), ())),
            preferred_element_type=jnp.float32) / sl


def _combine_body(co_ref, fo_ref, so_ref, st_ref, wo_ref, out_ref, acc):
    for h in range(H):
        gc = st_ref[:, h:h + 1]
        gf = st_ref[:, H + h:H + h + 1]
        gs = st_ref[:, 2 * H + h:2 * H + h + 1]
        acc[:, h * DH:(h + 1) * DH] = (
            gc * co_ref[h] + gf * fo_ref[h] + gs * so_ref[h])
    out_ref[...] = jnp.dot(acc[...].astype(jnp.bfloat16), wo_ref[...],
                           preferred_element_type=jnp.float32)


@jax.jit
def _run(x, gamma, Wq, Wk, Wv, k_pos, v_pos, mem_kv, kW1, kb1, kW2, kb2,
         vW1, vb1, vW2, vb2, Wcomb, bcomb, Wo):
    x2 = x[0]
    wc_r = Wcomb.reshape(DIM, H, 3).transpose(0, 2, 1).reshape(DIM, 3 * H)
    bc_r = bcomb.reshape(H, 3).transpose(1, 0).reshape(1, 3 * H)

    full = lambda shape: pl.BlockSpec(shape, lambda i: (0,) * len(shape))
    q2, k2, v2, strat = pl.pallas_call(
        _proj_body,
        grid=(NT,),
        in_specs=[
            pl.BlockSpec((TQ, DIM), lambda i: (i, 0)),
            full((1, DIM)), full((DIM, H * DH)), full((DIM, KVH * DH)),
            full((DIM, KVH * DH)), full((DIM, 3 * H)), full((1, 3 * H)),
        ],
        out_specs=[
            pl.BlockSpec((TQ, H * DH), lambda i: (i, 0)),
            pl.BlockSpec((TQ, KVH * DH), lambda i: (i, 0)),
            pl.BlockSpec((TQ, KVH * DH), lambda i: (i, 0)),
            pl.BlockSpec((TQ, 3 * H), lambda i: (i, 0)),
        ],
        out_shape=[
            jax.ShapeDtypeStruct((N, H * DH), jnp.float32),
            jax.ShapeDtypeStruct((N, KVH * DH), jnp.float32),
            jax.ShapeDtypeStruct((N, KVH * DH), jnp.float32),
            jax.ShapeDtypeStruct((N, 3 * H), jnp.float32),
        ],
    )(x2, gamma.reshape(1, DIM), Wq, Wk, Wv, wc_r, bc_r)

    kbf = (k2.reshape(W, CBS, KVH, DH).transpose(2, 0, 1, 3)
           .reshape(KVH, W, CD) + k_pos.reshape(KVH, 1, CD)).reshape(KVH * W, CD)
    vbf = (v2.reshape(W, CBS, KVH, DH).transpose(2, 0, 1, 3)
           .reshape(KVH, W, CD) + v_pos.reshape(KVH, 1, CD)).reshape(KVH * W, CD)

    ckm, cvm = pl.pallas_call(
        _compress_body,
        grid=(NHT,),
        in_specs=[
            full((KVH * W, CD)),
            pl.BlockSpec((CD, HT), lambda j: (0, j)),
            pl.BlockSpec((1, HT), lambda j: (0, j)),
            pl.BlockSpec((HT, DH), lambda j: (j, 0)),
            full((1, DH)),
            full((KVH * W, CD)),
            pl.BlockSpec((CD, HT), lambda j: (0, j)),
            pl.BlockSpec((1, HT), lambda j: (0, j)),
            pl.BlockSpec((HT, DH), lambda j: (j, 0)),
            full((1, DH)),
        ],
        out_specs=[
            pl.BlockSpec((KVH * W, DH), lambda j: (0, 0)),
            pl.BlockSpec((KVH * W, DH), lambda j: (0, 0)),
        ],
        out_shape=[
            jax.ShapeDtypeStruct((KVH * W, DH), jnp.float32),
            jax.ShapeDtypeStruct((KVH * W, DH), jnp.float32),
        ],
        compiler_params=pltpu.CompilerParams(
            dimension_semantics=("arbitrary",)),
    )(kbf, kW1, kb1.reshape(1, CD), kW2, kb2.reshape(1, DH),
      vbf, vW1, vb1.reshape(1, CD), vW2, vb2.reshape(1, DH))

    ckf = jnp.concatenate(
        [jnp.broadcast_to(mem_kv[0], (KVH, 1, DH)),
         ckm.reshape(KVH, W, DH)], axis=1)
    cvf = jnp.concatenate(
        [jnp.broadcast_to(mem_kv[1], (KVH, 1, DH)),
         cvm.reshape(KVH, W, DH)], axis=1)

    qg = q2.reshape(N, H, DH).transpose(1, 0, 2).reshape(KVH, G, N, DH)
    kh = k2.reshape(N, KVH, DH).transpose(1, 0, 2)
    vh = v2.reshape(N, KVH, DH).transpose(1, 0, 2)

    cout, imp = pl.pallas_call(
        _cattn_body,
        grid=(KVH, NT),
        in_specs=[
            pl.BlockSpec((1, G, TQ, DH), lambda h, i: (h, 0, i, 0)),
            pl.BlockSpec((1, W + 1, DH), lambda h, i: (h, 0, 0)),
            pl.BlockSpec((1, W + 1, DH), lambda h, i: (h, 0, 0)),
        ],
        out_specs=[
            pl.BlockSpec((1, G, TQ, DH), lambda h, i: (h, 0, i, 0)),
            pl.BlockSpec((1, TQ, W), lambda h, i: (h, i, 0)),
        ],
        out_shape=[
            jax.ShapeDtypeStruct((KVH, G, N, DH), jnp.float32),
            jax.ShapeDtypeStruct((KVH, N, W), jnp.float32),
        ],
    )(qg, ckf, cvf)

    sel = pl.pallas_call(
        _sel_body,
        grid=(KVH,),
        in_specs=[pl.BlockSpec((1, N, W), lambda h: (h, 0, 0))],
        out_specs=pl.BlockSpec((1, N, W), lambda h: (h, 0, 0)),
        out_shape=jax.ShapeDtypeStruct((KVH, N, W), jnp.float32),
    )(imp)

    em_full = (jnp.arange(W)[:, None]
               == jnp.arange(N)[None, :] // CBS).astype(jnp.float32)
    fparts, sparts = [], []
    NSPLIT = 2
    for tile0 in range(0, NT, NSPLIT):
        nw = (tile0 + NSPLIT) * TQ
        fp_, sp_ = pl.pallas_call(
            functools.partial(_attn_body, tile0=tile0, nw=nw),
            grid=(KVH, NSPLIT),
            in_specs=[
                pl.BlockSpec((1, G, TQ, DH),
                             lambda h, i, t0=tile0: (h, 0, t0 + i, 0)),
                pl.BlockSpec((1, nw, DH), lambda h, i: (h, 0, 0)),
                pl.BlockSpec((1, nw, DH), lambda h, i: (h, 0, 0)),
                pl.BlockSpec((1, TQ, W),
                             lambda h, i, t0=tile0: (h, t0 + i, 0)),
                pl.BlockSpec((nw // CBS, nw), lambda h, i: (0, 0)),
            ],
            out_specs=[
                pl.BlockSpec((1, G, TQ, DH), lambda h, i: (h, 0, i, 0)),
                pl.BlockSpec((1, G, TQ, DH), lambda h, i: (h, 0, i, 0)),
            ],
            out_shape=[
                jax.ShapeDtypeStruct((KVH, G, NSPLIT * TQ, DH), jnp.float32),
                jax.ShapeDtypeStruct((KVH, G, NSPLIT * TQ, DH), jnp.float32),
            ],
        )(qg, kh, vh, sel, em_full)
        fparts.append(fp_)
        sparts.append(sp_)
    fout = jnp.concatenate(fparts, axis=2)
    sout = jnp.concatenate(sparts, axis=2)

    out = pl.pallas_call(
        _combine_body,
        grid=(NT,),
        in_specs=[
            pl.BlockSpec((H, TQ, DH), lambda i: (0, i, 0)),
            pl.BlockSpec((H, TQ, DH), lambda i: (0, i, 0)),
            pl.BlockSpec((H, TQ, DH), lambda i: (0, i, 0)),
            pl.BlockSpec((TQ, 3 * H), lambda i: (i, 0)),
            full((H * DH, DIM)),
        ],
        out_specs=pl.BlockSpec((TQ, DIM), lambda i: (i, 0)),
        out_shape=jax.ShapeDtypeStruct((N, DIM), jnp.float32),
        scratch_shapes=[pltpu.VMEM((TQ, H * DH), jnp.float32)],
    )(cout.reshape(H, N, DH), fout.reshape(H, N, DH),
      sout.reshape(H, N, DH), strat, Wo.astype(jnp.bfloat16))

    return out[None]


def kernel(x, gamma, Wq, Wk, Wv, k_pos, v_pos, mem_kv, kW1, kb1, kW2, kb2,
           vW1, vb1, vW2, vb2, Wcomb, bcomb, Wo):
    return _run(x, gamma, Wq, Wk, Wv, k_pos, v_pos, mem_kv, kW1, kb1, kW2,
                kb2, vW1, vb1, vW2, vb2, Wcomb, bcomb, Wo)


# speedup vs baseline: 1.0067x; 1.0067x over previous
import functools

import jax
import jax.numpy as jnp
from jax.experimental import pallas as pl
from jax.experimental.pallas import tpu as pltpu

B, N, DIM = 1, 2048, 2048
H, KVH, DH = 16, 4, 128
G = H // KVH
CBS = 32
SBS = 32
NSEL = 16
WIN = 64
W = N // CBS
CD = CBS * DH
TQ = 128
NT = N // TQ
HT = 512
NHT = CD // HT
SCALE = DH ** -0.5


def _proj_body(x_ref, g_ref, wq_ref, wk_ref, wv_ref, wc_ref, bc_ref,
               q_ref, k_ref, v_ref, st_ref):
    xt = x_ref[...]
    ms = jnp.mean(xt * xt, axis=-1, keepdims=True)
    xn = xt * jax.lax.rsqrt(ms + 1e-6) * g_ref[...]
    q_ref[...] = jnp.dot(xn, wq_ref[...], preferred_element_type=jnp.float32)
    k_ref[...] = jnp.dot(xn, wk_ref[...], preferred_element_type=jnp.float32)
    v_ref[...] = jnp.dot(xn, wv_ref[...], preferred_element_type=jnp.float32)
    st_ref[...] = jax.nn.sigmoid(
        jnp.dot(xn, wc_ref[...], preferred_element_type=jnp.float32)
        + bc_ref[...])


def _compress_body(kb_ref, kw1_ref, kb1_ref, kw2_ref, kb2_ref,
                   vb_ref, vw1_ref, vb1_ref, vw2_ref, vb2_ref,
                   ck_ref, cv_ref):
    j = pl.program_id(0)

    @pl.when(j == 0)
    def _():
        ck_ref[...] = jnp.broadcast_to(kb2_ref[...], ck_ref.shape)
        cv_ref[...] = jnp.broadcast_to(vb2_ref[...], cv_ref.shape)

    hk = jnp.maximum(
        jnp.dot(kb_ref[...], kw1_ref[...], preferred_element_type=jnp.float32)
        + kb1_ref[...], 0.0)
    ck_ref[...] += jnp.dot(hk, kw2_ref[...], preferred_element_type=jnp.float32)
    hv = jnp.maximum(
        jnp.dot(vb_ref[...], vw1_ref[...], preferred_element_type=jnp.float32)
        + vb1_ref[...], 0.0)
    cv_ref[...] += jnp.dot(hv, vw2_ref[...], preferred_element_type=jnp.float32)


def _cattn_body(q_ref, ck_ref, cv_ref, co_ref, imp_ref):
    i = pl.program_id(1)
    qt = q_ref[...].reshape(G * TQ, DH)
    ck = ck_ref[...].reshape(W + 1, DH)
    cv = cv_ref[...].reshape(W + 1, DH)
    sim = jax.lax.dot_general(qt, ck, (((1,), (1,)), ((), ())),
                              preferred_element_type=jnp.float32) * SCALE
    qp = i * TQ + jax.lax.rem(
        jax.lax.broadcasted_iota(jnp.int32, (G * TQ, W + 1), 0), TQ)
    col = jax.lax.broadcasted_iota(jnp.int32, (G * TQ, W + 1), 1)
    vis = qp >= col * CBS - 1
    s = jnp.where(vis, sim, -1e30)
    m = jnp.max(s, axis=-1, keepdims=True)
    e = jnp.exp(s - m)
    p = e / jnp.sum(e, axis=-1, keepdims=True)
    co_ref[...] = jax.lax.dot_general(
        p.astype(jnp.bfloat16), cv.astype(jnp.bfloat16),
        (((1,), (0,)), ((), ())),
        preferred_element_type=jnp.float32).reshape(1, G, TQ, DH)
    imp_ref[...] = jnp.mean(p[:, 1:].reshape(G, TQ, W),
                            axis=0).reshape(1, TQ, W)


def _sel_body(imp_ref, sel_ref):
    imp = imp_ref[...].reshape(N, W)
    cur = imp
    for _ in range(NSEL - 1):
        mx = jnp.max(cur, axis=-1, keepdims=True)
        cur = jnp.where(cur >= mx, -1.0, cur)
    thr = jnp.max(cur, axis=-1, keepdims=True)
    qp = jax.lax.broadcasted_iota(jnp.int32, (N, W), 0)
    blk = jax.lax.broadcasted_iota(jnp.int32, (N, W), 1)
    sel = (imp >= thr) | (blk == qp // SBS)
    sel_ref[...] = sel.astype(jnp.float32).reshape(1, N, W)


def _attn_body(q_ref, k_ref, v_ref, sel_ref, fo_ref, so_ref, *, tile0, nw):
    i = tile0 + pl.program_id(1)
    wb = nw // CBS
    selb = sel_ref[...].reshape(TQ, W)[:, :wb]
    em = (jax.lax.broadcasted_iota(jnp.int32, (wb, nw), 0)
          == jax.lax.broadcasted_iota(jnp.int32, (wb, nw), 1) // CBS
          ).astype(jnp.float32)
    selt = jax.lax.dot_general(selb, em, (((1,), (0,)), ((), ())),
                               preferred_element_type=jnp.float32) > 0.5
    qp = i * TQ + jax.lax.broadcasted_iota(jnp.int32, (TQ, nw), 0)
    jp = jax.lax.broadcasted_iota(jnp.int32, (TQ, nw), 1)
    causal = qp >= jp
    fmask = selt & causal
    cs = jnp.maximum(i - 1, 0) * TQ
    qps = i * TQ + jax.lax.broadcasted_iota(jnp.int32, (TQ, 2 * TQ), 0)
    jps = cs + jax.lax.broadcasted_iota(jnp.int32, (TQ, 2 * TQ), 1)
    smask = (qps >= jps) & (qps - jps <= WIN)
    ksw = k_ref[0, pl.ds(cs, 2 * TQ), :].astype(jnp.bfloat16)
    vsw = v_ref[0, pl.ds(cs, 2 * TQ), :].astype(jnp.bfloat16)
    kt = k_ref[...].reshape(nw, DH).astype(jnp.bfloat16)
    vt = v_ref[...].reshape(nw, DH).astype(jnp.bfloat16)
    for g in range(G):
        qg = q_ref[0, g].astype(jnp.bfloat16)
        s = jax.lax.dot_general(qg, kt, (((1,), (1,)), ((), ())),
                                preferred_element_type=jnp.float32) * SCALE
        fs = jnp.where(fmask, s, -1e30)
        fm = jnp.max(fs, axis=-1, keepdims=True)
        fe = jnp.exp(fs - fm)
        fl = jnp.sum(fe, axis=-1, keepdims=True)
        fo_ref[0, g] = jax.lax.dot_general(
            fe.astype(jnp.bfloat16), vt, (((1,), (0,)), ((), ())),
            preferred_element_type=jnp.float32) / fl
        ssw = jax.lax.dot_general(qg, ksw, (((1,), (1,)), ((), ())),
                                  preferred_element_type=jnp.float32) * SCALE
        ss = jnp.where(smask, ssw, -1e30)
        sm = jnp.max(ss, axis=-1, keepdims=True)
        se = jnp.exp(ss - sm)
        sl = jnp.sum(se, axis=-1, keepdims=True)
        so_ref[0, g] = jax.lax.dot_general(
            se.astype(jnp.bfloat16), vsw, (((1,), (0,)), ((), ())),
            preferred_element_type=jnp.float32) / sl


def _combine_body(co_ref, fo_ref, so_ref, st_ref, wo_ref, out_ref, acc):
    for h in range(H):
        gc = st_ref[:, h:h + 1]
        gf = st_ref[:, H + h:H + h + 1]
        gs = st_ref[:, 2 * H + h:2 * H + h + 1]
        acc[:, h * DH:(h + 1) * DH] = (
            gc * co_ref[h] + gf * fo_ref[h] + gs * so_ref[h])
    out_ref[...] = jnp.dot(acc[...].astype(jnp.bfloat16), wo_ref[...],
                           preferred_element_type=jnp.float32)


@jax.jit
def _run(x, gamma, Wq, Wk, Wv, k_pos, v_pos, mem_kv, kW1, kb1, kW2, kb2,
         vW1, vb1, vW2, vb2, Wcomb, bcomb, Wo):
    x2 = x[0]
    wc_r = Wcomb.reshape(DIM, H, 3).transpose(0, 2, 1).reshape(DIM, 3 * H)
    bc_r = bcomb.reshape(H, 3).transpose(1, 0).reshape(1, 3 * H)

    full = lambda shape: pl.BlockSpec(shape, lambda i: (0,) * len(shape))
    q2, k2, v2, strat = pl.pallas_call(
        _proj_body,
        grid=(NT,),
        in_specs=[
            pl.BlockSpec((TQ, DIM), lambda i: (i, 0)),
            full((1, DIM)), full((DIM, H * DH)), full((DIM, KVH * DH)),
            full((DIM, KVH * DH)), full((DIM, 3 * H)), full((1, 3 * H)),
        ],
        out_specs=[
            pl.BlockSpec((TQ, H * DH), lambda i: (i, 0)),
            pl.BlockSpec((TQ, KVH * DH), lambda i: (i, 0)),
            pl.BlockSpec((TQ, KVH * DH), lambda i: (i, 0)),
            pl.BlockSpec((TQ, 3 * H), lambda i: (i, 0)),
        ],
        out_shape=[
            jax.ShapeDtypeStruct((N, H * DH), jnp.float32),
            jax.ShapeDtypeStruct((N, KVH * DH), jnp.float32),
            jax.ShapeDtypeStruct((N, KVH * DH), jnp.float32),
            jax.ShapeDtypeStruct((N, 3 * H), jnp.float32),
        ],
    )(x2, gamma.reshape(1, DIM), Wq, Wk, Wv, wc_r, bc_r)

    kbf = (k2.reshape(W, CBS, KVH, DH).transpose(2, 0, 1, 3)
           .reshape(KVH, W, CD) + k_pos.reshape(KVH, 1, CD)).reshape(KVH * W, CD)
    vbf = (v2.reshape(W, CBS, KVH, DH).transpose(2, 0, 1, 3)
           .reshape(KVH, W, CD) + v_pos.reshape(KVH, 1, CD)).reshape(KVH * W, CD)

    ckm, cvm = pl.pallas_call(
        _compress_body,
        grid=(NHT,),
        in_specs=[
            full((KVH * W, CD)),
            pl.BlockSpec((CD, HT), lambda j: (0, j)),
            pl.BlockSpec((1, HT), lambda j: (0, j)),
            pl.BlockSpec((HT, DH), lambda j: (j, 0)),
            full((1, DH)),
            full((KVH * W, CD)),
            pl.BlockSpec((CD, HT), lambda j: (0, j)),
            pl.BlockSpec((1, HT), lambda j: (0, j)),
            pl.BlockSpec((HT, DH), lambda j: (j, 0)),
            full((1, DH)),
        ],
        out_specs=[
            pl.BlockSpec((KVH * W, DH), lambda j: (0, 0)),
            pl.BlockSpec((KVH * W, DH), lambda j: (0, 0)),
        ],
        out_shape=[
            jax.ShapeDtypeStruct((KVH * W, DH), jnp.float32),
            jax.ShapeDtypeStruct((KVH * W, DH), jnp.float32),
        ],
        compiler_params=pltpu.CompilerParams(
            dimension_semantics=("arbitrary",)),
    )(kbf, kW1, kb1.reshape(1, CD), kW2, kb2.reshape(1, DH),
      vbf, vW1, vb1.reshape(1, CD), vW2, vb2.reshape(1, DH))

    ckf = jnp.concatenate(
        [jnp.broadcast_to(mem_kv[0], (KVH, 1, DH)),
         ckm.reshape(KVH, W, DH)], axis=1)
    cvf = jnp.concatenate(
        [jnp.broadcast_to(mem_kv[1], (KVH, 1, DH)),
         cvm.reshape(KVH, W, DH)], axis=1)

    qg = q2.reshape(N, H, DH).transpose(1, 0, 2).reshape(KVH, G, N, DH)
    kh = k2.reshape(N, KVH, DH).transpose(1, 0, 2)
    vh = v2.reshape(N, KVH, DH).transpose(1, 0, 2)

    cout, imp = pl.pallas_call(
        _cattn_body,
        grid=(KVH, NT),
        in_specs=[
            pl.BlockSpec((1, G, TQ, DH), lambda h, i: (h, 0, i, 0)),
            pl.BlockSpec((1, W + 1, DH), lambda h, i: (h, 0, 0)),
            pl.BlockSpec((1, W + 1, DH), lambda h, i: (h, 0, 0)),
        ],
        out_specs=[
            pl.BlockSpec((1, G, TQ, DH), lambda h, i: (h, 0, i, 0)),
            pl.BlockSpec((1, TQ, W), lambda h, i: (h, i, 0)),
        ],
        out_shape=[
            jax.ShapeDtypeStruct((KVH, G, N, DH), jnp.float32),
            jax.ShapeDtypeStruct((KVH, N, W), jnp.float32),
        ],
    )(qg, ckf, cvf)

    sel = pl.pallas_call(
        _sel_body,
        grid=(KVH,),
        in_specs=[pl.BlockSpec((1, N, W), lambda h: (h, 0, 0))],
        out_specs=pl.BlockSpec((1, N, W), lambda h: (h, 0, 0)),
        out_shape=jax.ShapeDtypeStruct((KVH, N, W), jnp.float32),
    )(imp)

    fparts, sparts = [], []
    NSPLIT = 2
    for tile0 in range(0, NT, NSPLIT):
        nw = (tile0 + NSPLIT) * TQ
        fp_, sp_ = pl.pallas_call(
            functools.partial(_attn_body, tile0=tile0, nw=nw),
            grid=(KVH, NSPLIT),
            in_specs=[
                pl.BlockSpec((1, G, TQ, DH),
                             lambda h, i, t0=tile0: (h, 0, t0 + i, 0)),
                pl.BlockSpec((1, nw, DH), lambda h, i: (h, 0, 0)),
                pl.BlockSpec((1, nw, DH), lambda h, i: (h, 0, 0)),
                pl.BlockSpec((1, TQ, W),
                             lambda h, i, t0=tile0: (h, t0 + i, 0)),
            ],
            out_specs=[
                pl.BlockSpec((1, G, TQ, DH), lambda h, i: (h, 0, i, 0)),
                pl.BlockSpec((1, G, TQ, DH), lambda h, i: (h, 0, i, 0)),
            ],
            out_shape=[
                jax.ShapeDtypeStruct((KVH, G, NSPLIT * TQ, DH), jnp.float32),
                jax.ShapeDtypeStruct((KVH, G, NSPLIT * TQ, DH), jnp.float32),
            ],
        )(qg, kh, vh, sel)
        fparts.append(fp_)
        sparts.append(sp_)
    fout = jnp.concatenate(fparts, axis=2)
    sout = jnp.concatenate(sparts, axis=2)

    out = pl.pallas_call(
        _combine_body,
        grid=(NT,),
        in_specs=[
            pl.BlockSpec((H, TQ, DH), lambda i: (0, i, 0)),
            pl.BlockSpec((H, TQ, DH), lambda i: (0, i, 0)),
            pl.BlockSpec((H, TQ, DH), lambda i: (0, i, 0)),
            pl.BlockSpec((TQ, 3 * H), lambda i: (i, 0)),
            full((H * DH, DIM)),
        ],
        out_specs=pl.BlockSpec((TQ, DIM), lambda i: (i, 0)),
        out_shape=jax.ShapeDtypeStruct((N, DIM), jnp.float32),
        scratch_shapes=[pltpu.VMEM((TQ, H * DH), jnp.float32)],
    )(cout.reshape(H, N, DH), fout.reshape(H, N, DH),
      sout.reshape(H, N, DH), strat, Wo.astype(jnp.bfloat16))

    return out[None]


def kernel(x, gamma, Wq, Wk, Wv, k_pos, v_pos, mem_kv, kW1, kb1, kW2, kb2,
           vW1, vb1, vW2, vb2, Wcomb, bcomb, Wo):
    return _run(x, gamma, Wq, Wk, Wv, k_pos, v_pos, mem_kv, kW1, kb1, kW2,
                kb2, vW1, vb1, vW2, vb2, Wcomb, bcomb, Wo)
